# baseline (speedup 1.0000x reference)
import functools

import jax
import jax.numpy as jnp
from jax import lax
from jax.experimental import pallas as pl
from jax.experimental.pallas import tpu as pltpu
from jax.experimental.pallas import tpu_sc as plsc

E = 8
D = 1024
H = 512
T = 2048
K = 2
TK = T * K

B = 256
NB = (TK + E * (B - 1) + B - 1) // B
NPAD = NB * B

NC = 2
NS = 16
NW = NC * NS

_CTOK = T // NW
_CCHUNK = 16


@functools.cache
def _sc_combine():
    mesh = plsc.VectorSubcoreMesh(core_axis_name="c", subcore_axis_name="s")

    @functools.partial(
        pl.kernel,
        out_type=jax.ShapeDtypeStruct((T, D), jnp.float32),
        mesh=mesh,
        scratch_types=[
            pltpu.VMEM((_CTOK // _CCHUNK, _CCHUNK), jnp.int32),
            pltpu.VMEM((_CTOK // _CCHUNK, _CCHUNK), jnp.int32),
            pltpu.VMEM((_CTOK, 16), jnp.float32),
            pltpu.VMEM((_CTOK, 16), jnp.float32),
            pltpu.VMEM((2, _CCHUNK, D), jnp.float32),
            pltpu.VMEM((2, _CCHUNK, D), jnp.float32),
            pltpu.VMEM((2, _CCHUNK, D), jnp.float32),
            pltpu.SemaphoreType.DMA((2,)),
            pltpu.SemaphoreType.DMA((2,)),
            pltpu.SemaphoreType.DMA,
        ],
    )
    def sc_combine(rows_hbm, pos_a_hbm, pos_b_hbm, w0_hbm, w1_hbm, y_hbm,
                   idx_a, idx_b, w0v, w1v, buf_a, buf_b, buf_o,
                   sem_a, sem_b, sem_o):
        wid = lax.axis_index("s") * NC + lax.axis_index("c")
        base = wid * _CTOK
        nch = _CTOK // _CCHUNK
        pltpu.sync_copy(pos_a_hbm.at[wid], idx_a)
        pltpu.sync_copy(pos_b_hbm.at[wid], idx_b)
        pltpu.sync_copy(w0_hbm.at[pl.ds(base, _CTOK)], w0v)
        pltpu.sync_copy(w1_hbm.at[pl.ds(base, _CTOK)], w1v)

        def fire(c):
            slot = c % 2
            return (pltpu.async_copy(rows_hbm.at[idx_a.at[c]], buf_a.at[slot],
                                     sem_a.at[slot]),
                    pltpu.async_copy(rows_hbm.at[idx_b.at[c]], buf_b.at[slot],
                                     sem_b.at[slot]))

        cps = fire(0)
        wr = None
        for c in range(nch):
            slot = c % 2
            nxt = fire(c + 1) if c + 1 < nch else None
            cps[0].wait()
            cps[1].wait()

            def _row_add(r, carry):
                w0r = w0v[c * _CCHUNK + r, :]
                w1r = w1v[c * _CCHUNK + r, :]
                for j in range(D // 16):
                    sl = pl.ds(j * 16, 16)
                    buf_o[slot, r, sl] = (buf_a[slot, r, sl] * w0r
                                          + buf_b[slot, r, sl] * w1r)
                return carry

            lax.fori_loop(0, _CCHUNK, _row_add, 0)

            if wr is not None:
                wr.wait()
            wr = pltpu.async_copy(
                buf_o.at[slot], y_hbm.at[pl.ds(base + c * _CCHUNK, _CCHUNK)],
                sem_o)
            cps = nxt
        wr.wait()

    return sc_combine


def _expert_block(be_ref, x_ref, dst_ref, w1_ref, w2_ref, out_ref,
                  x16_ref, w1c_ref, w2c_ref):
    b = pl.program_id(0)

    @pl.when(b == 0)
    def _():
        x16_ref[...] = x_ref[...].astype(jnp.bfloat16)

    prev = be_ref[jnp.maximum(b - 1, 0)]
    changed = jnp.logical_or(b == 0, be_ref[b] != prev)

    @pl.when(changed)
    def _():
        w1c_ref[...] = w1_ref[0].astype(jnp.bfloat16)
        w2c_ref[...] = w2_ref[0].astype(jnp.bfloat16)

    rowid = B * b + lax.broadcasted_iota(jnp.int32, (B, 1), 0)
    d0 = dst_ref[0, 0:1, :]
    d1 = dst_ref[0, 1:2, :]
    p = ((rowid == d0) | (rowid == d1)).astype(jnp.bfloat16)
    xb = jnp.dot(p, x16_ref[...],
                 preferred_element_type=jnp.float32).astype(jnp.bfloat16)
    h = jnp.dot(xb, w1c_ref[...], preferred_element_type=jnp.float32)
    up = h[:, :H]
    gate = h[:, H:]
    act = up * (gate * jax.lax.logistic(gate))
    out_ref[...] = jnp.dot(act.astype(jnp.bfloat16), w2c_ref[...],
                           preferred_element_type=jnp.float32)


def _tc_experts(block_expert, x, dstT3, W1, W2):
    grid_spec = pltpu.PrefetchScalarGridSpec(
        num_scalar_prefetch=1,
        grid=(NB,),
        in_specs=[
            pl.BlockSpec((T, D), lambda b, be: (0, 0)),
            pl.BlockSpec((1, K, T), lambda b, be: (0, 0, 0)),
            pl.BlockSpec((1, D, 2 * H), lambda b, be: (be[b], 0, 0)),
            pl.BlockSpec((1, H, D), lambda b, be: (be[b], 0, 0)),
        ],
        out_specs=pl.BlockSpec((B, D), lambda b, be: (b, 0)),
        scratch_shapes=[
            pltpu.VMEM((T, D), jnp.bfloat16),
            pltpu.VMEM((D, 2 * H), jnp.bfloat16),
            pltpu.VMEM((H, D), jnp.bfloat16),
        ],
    )
    return pl.pallas_call(
        _expert_block,
        grid_spec=grid_spec,
        out_shape=jax.ShapeDtypeStruct((NPAD, D), jnp.float32),
    )(block_expert, x, dstT3, W1, W2)


def _routing_metadata(indices):
    e_flat = indices.reshape(TK).astype(jnp.int32)
    onehot = (e_flat[:, None] == jnp.arange(E, dtype=jnp.int32)[None, :])
    ohi = onehot.astype(jnp.int32)
    rank = jnp.sum((jnp.cumsum(ohi, axis=0) - ohi) * ohi, axis=1)
    counts = jnp.sum(ohi, axis=0)
    padded_counts = ((counts + B - 1) // B) * B
    padded_ends = jnp.cumsum(padded_counts)
    padded_starts = padded_ends - padded_counts
    dest = (padded_starts[e_flat] + rank).reshape(T, K)
    block_expert = jnp.searchsorted(
        padded_ends, jnp.arange(NB, dtype=jnp.int32) * B, side="right")
    block_expert = jnp.minimum(block_expert, E - 1).astype(jnp.int32)
    return dest, block_expert


def kernel(x, weights, indices, W1, W2):
    dest, block_expert = _routing_metadata(indices)
    out_sorted = _tc_experts(block_expert, x, dest.T.reshape(1, K, T), W1, W2)
    w0b = jnp.broadcast_to(weights[:, 0:1], (T, 16))
    w1b = jnp.broadcast_to(weights[:, 1:2], (T, 16))
    pos_a = dest[:, 0].reshape(NW, _CTOK // _CCHUNK, _CCHUNK)
    pos_b = dest[:, 1].reshape(NW, _CTOK // _CCHUNK, _CCHUNK)
    return _sc_combine()(out_sorted, pos_a, pos_b, w0b, w1b)


# speedup vs baseline: 1.0292x; 1.0292x over previous
import functools

import jax
import jax.numpy as jnp
from jax import lax
from jax.experimental import pallas as pl
from jax.experimental.pallas import tpu as pltpu
from jax.experimental.pallas import tpu_sc as plsc

E = 8
D = 1024
H = 512
T = 2048
K = 2
TK = T * K

B = 256
NB = (TK + E * (B - 1) + B - 1) // B
NPAD = NB * B

NC = 2
NS = 16
NW = NC * NS

_CTOK = T // NW
_CCHUNK = 16


@functools.cache
def _sc_combine():
    mesh = plsc.VectorSubcoreMesh(core_axis_name="c", subcore_axis_name="s")

    @functools.partial(
        pl.kernel,
        out_type=jax.ShapeDtypeStruct((T, D), jnp.float32),
        mesh=mesh,
        scratch_types=[
            pltpu.VMEM((_CTOK // _CCHUNK, _CCHUNK), jnp.int32),
            pltpu.VMEM((_CTOK // _CCHUNK, _CCHUNK), jnp.int32),
            pltpu.VMEM((_CTOK, 16), jnp.float32),
            pltpu.VMEM((_CTOK, 16), jnp.float32),
            pltpu.VMEM((2, _CCHUNK, D), jnp.float32),
            pltpu.VMEM((2, _CCHUNK, D), jnp.float32),
            pltpu.VMEM((2, _CCHUNK, D), jnp.float32),
            pltpu.SemaphoreType.DMA((2,)),
            pltpu.SemaphoreType.DMA((2,)),
            pltpu.SemaphoreType.DMA,
        ],
    )
    def sc_combine(rows_hbm, pos_a_hbm, pos_b_hbm, w0_hbm, w1_hbm, y_hbm,
                   idx_a, idx_b, w0v, w1v, buf_a, buf_b, buf_o,
                   sem_a, sem_b, sem_o):
        wid = lax.axis_index("s") * NC + lax.axis_index("c")
        base = wid * _CTOK
        nch = _CTOK // _CCHUNK
        pltpu.sync_copy(pos_a_hbm.at[wid], idx_a)
        pltpu.sync_copy(pos_b_hbm.at[wid], idx_b)
        pltpu.sync_copy(w0_hbm.at[pl.ds(base, _CTOK)], w0v)
        pltpu.sync_copy(w1_hbm.at[pl.ds(base, _CTOK)], w1v)

        def fire(c):
            slot = c % 2
            return (pltpu.async_copy(rows_hbm.at[idx_a.at[c]], buf_a.at[slot],
                                     sem_a.at[slot]),
                    pltpu.async_copy(rows_hbm.at[idx_b.at[c]], buf_b.at[slot],
                                     sem_b.at[slot]))

        cps = fire(0)
        wr = None
        for c in range(nch):
            slot = c % 2
            nxt = fire(c + 1) if c + 1 < nch else None
            cps[0].wait()
            cps[1].wait()

            def _row_add(r, carry):
                w0r = w0v[c * _CCHUNK + r, :]
                w1r = w1v[c * _CCHUNK + r, :]
                for j in range(D // 16):
                    sl = pl.ds(j * 16, 16)
                    buf_o[slot, r, sl] = (buf_a[slot, r, sl] * w0r
                                          + buf_b[slot, r, sl] * w1r)
                return carry

            lax.fori_loop(0, _CCHUNK, _row_add, 0)

            if wr is not None:
                wr.wait()
            wr = pltpu.async_copy(
                buf_o.at[slot], y_hbm.at[pl.ds(base + c * _CCHUNK, _CCHUNK)],
                sem_o)
            cps = nxt
        wr.wait()

    return sc_combine


def _expert_block(be_ref, x_ref, dst_ref, w1_ref, w2_ref, out_ref,
                  x16_ref, w1c_ref, w2c_ref):
    b = pl.program_id(0)

    @pl.when(b == 0)
    def _():
        x16_ref[...] = x_ref[...].astype(jnp.bfloat16)

    @pl.when(b < be_ref[NB])
    def _():
        prev = be_ref[jnp.maximum(b - 1, 0)]
        changed = jnp.logical_or(b == 0, be_ref[b] != prev)

        @pl.when(changed)
        def _():
            w1c_ref[...] = w1_ref[0].astype(jnp.bfloat16)
            w2c_ref[...] = w2_ref[0].astype(jnp.bfloat16)

        rowid = B * b + lax.broadcasted_iota(jnp.int32, (B, 1), 0)
        d0 = dst_ref[0, 0:1, :]
        d1 = dst_ref[0, 1:2, :]
        p = ((rowid == d0) | (rowid == d1)).astype(jnp.bfloat16)
        xb = jnp.dot(p, x16_ref[...],
                     preferred_element_type=jnp.float32).astype(jnp.bfloat16)
        h = jnp.dot(xb, w1c_ref[...], preferred_element_type=jnp.float32)
        up = h[:, :H]
        gate = h[:, H:]
        act = up * (gate * jax.lax.logistic(gate))
        out_ref[...] = jnp.dot(act.astype(jnp.bfloat16), w2c_ref[...],
                               preferred_element_type=jnp.float32)


def _tc_experts(block_expert, x, dstT3, W1, W2):
    grid_spec = pltpu.PrefetchScalarGridSpec(
        num_scalar_prefetch=1,
        grid=(NB,),
        in_specs=[
            pl.BlockSpec((T, D), lambda b, be: (0, 0)),
            pl.BlockSpec((1, K, T), lambda b, be: (0, 0, 0)),
            pl.BlockSpec((1, D, 2 * H), lambda b, be: (be[b], 0, 0)),
            pl.BlockSpec((1, H, D), lambda b, be: (be[b], 0, 0)),
        ],
        out_specs=pl.BlockSpec((B, D), lambda b, be: (b, 0)),
        scratch_shapes=[
            pltpu.VMEM((T, D), jnp.bfloat16),
            pltpu.VMEM((D, 2 * H), jnp.bfloat16),
            pltpu.VMEM((H, D), jnp.bfloat16),
        ],
    )
    return pl.pallas_call(
        _expert_block,
        grid_spec=grid_spec,
        out_shape=jax.ShapeDtypeStruct((NPAD, D), jnp.float32),
    )(block_expert, x, dstT3, W1, W2)


def _routing_metadata(indices):
    e_flat = indices.reshape(TK).astype(jnp.int32)
    onehot = (e_flat[:, None] == jnp.arange(E, dtype=jnp.int32)[None, :])
    ohi = onehot.astype(jnp.int32)
    rank = jnp.sum((jnp.cumsum(ohi, axis=0) - ohi) * ohi, axis=1)
    counts = jnp.sum(ohi, axis=0)
    padded_counts = ((counts + B - 1) // B) * B
    padded_ends = jnp.cumsum(padded_counts)
    padded_starts = padded_ends - padded_counts
    dest = (padded_starts[e_flat] + rank).reshape(T, K)
    block_expert = jnp.searchsorted(
        padded_ends, jnp.arange(NB, dtype=jnp.int32) * B, side="right")
    block_expert = jnp.minimum(block_expert, E - 1).astype(jnp.int32)
    nactive = padded_ends[E - 1] // B
    block_expert = jnp.concatenate([block_expert, nactive[None]])
    return dest, block_expert


def kernel(x, weights, indices, W1, W2):
    dest, block_expert = _routing_metadata(indices)
    out_sorted = _tc_experts(block_expert, x, dest.T.reshape(1, K, T), W1, W2)
    w0b = jnp.broadcast_to(weights[:, 0:1], (T, 16))
    w1b = jnp.broadcast_to(weights[:, 1:2], (T, 16))
    pos_a = dest[:, 0].reshape(NW, _CTOK // _CCHUNK, _CCHUNK)
    pos_b = dest[:, 1].reshape(NW, _CTOK // _CCHUNK, _CCHUNK)
    return _sc_combine()(out_sorted, pos_a, pos_b, w0b, w1b)


# speedup vs baseline: 1.1165x; 1.0848x over previous
import functools

import jax
import jax.numpy as jnp
from jax import lax
from jax.experimental import pallas as pl
from jax.experimental.pallas import tpu as pltpu
from jax.experimental.pallas import tpu_sc as plsc

E = 8
D = 1024
H = 512
T = 2048
K = 2
TK = T * K

B = 256
NB = (TK + E * (B - 1) + B - 1) // B
NPAD = NB * B

NC = 2
NS = 16
NW = NC * NS

_CTOK = T // NW
_CCHUNK = 16


@functools.cache
def _sc_combine():
    mesh = plsc.VectorSubcoreMesh(core_axis_name="c", subcore_axis_name="s")

    @functools.partial(
        pl.kernel,
        out_type=jax.ShapeDtypeStruct((T, D), jnp.float32),
        mesh=mesh,
        scratch_types=[
            pltpu.VMEM((_CTOK // _CCHUNK, _CCHUNK), jnp.int32),
            pltpu.VMEM((_CTOK // _CCHUNK, _CCHUNK), jnp.int32),
            pltpu.VMEM((_CTOK, 16), jnp.float32),
            pltpu.VMEM((_CTOK, 16), jnp.float32),
            pltpu.VMEM((2, _CCHUNK, D), jnp.float32),
            pltpu.VMEM((2, _CCHUNK, D), jnp.float32),
            pltpu.VMEM((2, _CCHUNK, D), jnp.float32),
            pltpu.SemaphoreType.DMA((2,)),
            pltpu.SemaphoreType.DMA((2,)),
            pltpu.SemaphoreType.DMA,
        ],
    )
    def sc_combine(rows_hbm, pos_a_hbm, pos_b_hbm, w0_hbm, w1_hbm, y_hbm,
                   idx_a, idx_b, w0v, w1v, buf_a, buf_b, buf_o,
                   sem_a, sem_b, sem_o):
        wid = lax.axis_index("s") * NC + lax.axis_index("c")
        base = wid * _CTOK
        nch = _CTOK // _CCHUNK
        pltpu.sync_copy(pos_a_hbm.at[wid], idx_a)
        pltpu.sync_copy(pos_b_hbm.at[wid], idx_b)
        pltpu.sync_copy(w0_hbm.at[pl.ds(base, _CTOK)], w0v)
        pltpu.sync_copy(w1_hbm.at[pl.ds(base, _CTOK)], w1v)

        def fire(c):
            slot = c % 2
            return (pltpu.async_copy(rows_hbm.at[idx_a.at[c]], buf_a.at[slot],
                                     sem_a.at[slot]),
                    pltpu.async_copy(rows_hbm.at[idx_b.at[c]], buf_b.at[slot],
                                     sem_b.at[slot]))

        cps = fire(0)
        wr = None
        for c in range(nch):
            slot = c % 2
            nxt = fire(c + 1) if c + 1 < nch else None
            cps[0].wait()
            cps[1].wait()

            def _row_add(r, carry):
                w0r = w0v[c * _CCHUNK + r, :]
                w1r = w1v[c * _CCHUNK + r, :]
                for j in range(D // 16):
                    sl = pl.ds(j * 16, 16)
                    buf_o[slot, r, sl] = (buf_a[slot, r, sl] * w0r
                                          + buf_b[slot, r, sl] * w1r)
                return carry

            lax.fori_loop(0, _CCHUNK, _row_add, 0)

            if wr is not None:
                wr.wait()
            wr = pltpu.async_copy(
                buf_o.at[slot], y_hbm.at[pl.ds(base + c * _CCHUNK, _CCHUNK)],
                sem_o)
            cps = nxt
        wr.wait()

    return sc_combine


def _expert_block(be_ref, x_ref, idx_ref, w1_ref, w2_ref,
                  out_ref, dest_out_ref, x16_ref, w1c_ref, w2c_ref, dst_s):
    b = pl.program_id(0)

    @pl.when(b == 0)
    def _():
        x16_ref[...] = x_ref[...].astype(jnp.bfloat16)
        idx = idx_ref[0]
        eids = lax.broadcasted_iota(jnp.int32, (E, 1), 0)
        oh0 = (idx[0:1, :] == eids).astype(jnp.int32)
        oh1 = (idx[1:2, :] == eids).astype(jnp.int32)
        g = oh0 + oh1
        s_inc = g
        sh = 1
        while sh < T:
            s_inc = s_inc + jnp.pad(s_inc, ((0, 0), (sh, 0)))[:, :T]
            sh *= 2
        s_exc = s_inc - g
        counts = s_inc[:, T - 1:T]
        padded = ((counts + B - 1) // B) * B
        pc = padded
        sh = 1
        while sh < E:
            pc = pc + jnp.pad(pc, ((sh, 0), (0, 0)))[:E, :]
            sh *= 2
        pstart = pc - padded
        d0 = (jnp.sum((s_exc + pstart) * oh0, axis=0, keepdims=True))
        d1 = (jnp.sum((s_exc + pstart) * oh1, axis=0, keepdims=True))
        dst = jnp.concatenate([d0, d1], axis=0)
        dst_s[...] = dst
        dest_out_ref[0] = dst

    @pl.when(b < be_ref[NB])
    def _():
        prev = be_ref[jnp.maximum(b - 1, 0)]
        changed = jnp.logical_or(b == 0, be_ref[b] != prev)

        @pl.when(changed)
        def _():
            w1c_ref[...] = w1_ref[0].astype(jnp.bfloat16)
            w2c_ref[...] = w2_ref[0].astype(jnp.bfloat16)

        rowid = B * b + lax.broadcasted_iota(jnp.int32, (B, 1), 0)
        d0 = dst_s[0:1, :]
        d1 = dst_s[1:2, :]
        p = ((rowid == d0) | (rowid == d1)).astype(jnp.bfloat16)
        xb = jnp.dot(p, x16_ref[...],
                     preferred_element_type=jnp.float32).astype(jnp.bfloat16)
        h = jnp.dot(xb, w1c_ref[...], preferred_element_type=jnp.float32)
        up = h[:, :H]
        gate = h[:, H:]
        act = up * (gate * jax.lax.logistic(gate))
        out_ref[...] = jnp.dot(act.astype(jnp.bfloat16), w2c_ref[...],
                               preferred_element_type=jnp.float32)


def _tc_experts(block_expert, x, idx3, W1, W2):
    grid_spec = pltpu.PrefetchScalarGridSpec(
        num_scalar_prefetch=1,
        grid=(NB,),
        in_specs=[
            pl.BlockSpec((T, D), lambda b, be: (0, 0)),
            pl.BlockSpec((1, K, T), lambda b, be: (0, 0, 0)),
            pl.BlockSpec((1, D, 2 * H), lambda b, be: (be[b], 0, 0)),
            pl.BlockSpec((1, H, D), lambda b, be: (be[b], 0, 0)),
        ],
        out_specs=[
            pl.BlockSpec((B, D), lambda b, be: (b, 0)),
            pl.BlockSpec((1, K, T), lambda b, be: (0, 0, 0)),
        ],
        scratch_shapes=[
            pltpu.VMEM((T, D), jnp.bfloat16),
            pltpu.VMEM((D, 2 * H), jnp.bfloat16),
            pltpu.VMEM((H, D), jnp.bfloat16),
            pltpu.VMEM((K, T), jnp.int32),
        ],
    )
    return pl.pallas_call(
        _expert_block,
        grid_spec=grid_spec,
        out_shape=[jax.ShapeDtypeStruct((NPAD, D), jnp.float32),
                   jax.ShapeDtypeStruct((1, K, T), jnp.int32)],
    )(block_expert, x, idx3, W1, W2)


def _block_experts(indices):
    e_flat = indices.reshape(TK).astype(jnp.int32)
    onehot = (e_flat[:, None] == jnp.arange(E, dtype=jnp.int32)[None, :])
    counts = jnp.sum(onehot.astype(jnp.int32), axis=0)
    padded_ends = jnp.cumsum(((counts + B - 1) // B) * B)
    block_expert = jnp.searchsorted(
        padded_ends, jnp.arange(NB, dtype=jnp.int32) * B, side="right")
    block_expert = jnp.minimum(block_expert, E - 1).astype(jnp.int32)
    nactive = padded_ends[E - 1] // B
    return jnp.concatenate([block_expert, nactive[None]])


def kernel(x, weights, indices, W1, W2):
    block_expert = _block_experts(indices)
    idx3 = indices.astype(jnp.int32).T.reshape(1, K, T)
    out_sorted, dest_out = _tc_experts(block_expert, x, idx3, W1, W2)
    w0b = jnp.broadcast_to(weights[:, 0:1], (T, 16))
    w1b = jnp.broadcast_to(weights[:, 1:2], (T, 16))
    pos_a = dest_out[0, 0].reshape(NW, _CTOK // _CCHUNK, _CCHUNK)
    pos_b = dest_out[0, 1].reshape(NW, _CTOK // _CCHUNK, _CCHUNK)
    return _sc_combine()(out_sorted, pos_a, pos_b, w0b, w1b)


# speedup vs baseline: 1.1595x; 1.0385x over previous
import functools

import jax
import jax.numpy as jnp
from jax import lax
from jax.experimental import pallas as pl
from jax.experimental.pallas import tpu as pltpu
from jax.experimental.pallas import tpu_sc as plsc

E = 8
D = 1024
H = 512
T = 2048
K = 2
TK = T * K

B = 256
NB = (TK + E * (B - 1) + B - 1) // B
NPAD = NB * B

NC = 2
NS = 16
NW = NC * NS

_CTOK = T // NW
_CCHUNK = 16


@functools.cache
def _sc_combine():
    mesh = plsc.VectorSubcoreMesh(core_axis_name="c", subcore_axis_name="s")

    @functools.partial(
        pl.kernel,
        out_type=jax.ShapeDtypeStruct((T, D), jnp.float32),
        mesh=mesh,
        scratch_types=[
            pltpu.VMEM((_CTOK,), jnp.int32),
            pltpu.VMEM((_CTOK,), jnp.int32),
            pltpu.VMEM((_CTOK, 16), jnp.float32),
            pltpu.VMEM((_CTOK, 16), jnp.float32),
            pltpu.VMEM((2, _CCHUNK, D), jnp.float32),
            pltpu.VMEM((2, _CCHUNK, D), jnp.float32),
            pltpu.VMEM((2, _CCHUNK, D), jnp.float32),
            pltpu.SemaphoreType.DMA((2,)),
            pltpu.SemaphoreType.DMA((2,)),
            pltpu.SemaphoreType.DMA,
        ],
    )
    def sc_combine(rows_hbm, pos_hbm, w_hbm, y_hbm,
                   idx_a, idx_b, w0v, w1v, buf_a, buf_b, buf_o,
                   sem_a, sem_b, sem_o):
        wid = lax.axis_index("s") * NC + lax.axis_index("c")
        base = wid * _CTOK
        nch = _CTOK // _CCHUNK
        pltpu.sync_copy(pos_hbm.at[0, 0, pl.ds(base, _CTOK)], idx_a)
        pltpu.sync_copy(pos_hbm.at[0, 1, pl.ds(base, _CTOK)], idx_b)
        pltpu.sync_copy(w_hbm.at[0, pl.ds(base, _CTOK)], w0v)
        pltpu.sync_copy(w_hbm.at[1, pl.ds(base, _CTOK)], w1v)

        def fire(c):
            slot = c % 2
            ia = idx_a.at[pl.ds(c * _CCHUNK, _CCHUNK)]
            ib = idx_b.at[pl.ds(c * _CCHUNK, _CCHUNK)]
            return (pltpu.async_copy(rows_hbm.at[ia], buf_a.at[slot],
                                     sem_a.at[slot]),
                    pltpu.async_copy(rows_hbm.at[ib], buf_b.at[slot],
                                     sem_b.at[slot]))

        cps = fire(0)
        wr = None
        for c in range(nch):
            slot = c % 2
            nxt = fire(c + 1) if c + 1 < nch else None
            cps[0].wait()
            cps[1].wait()

            def _row_add(r, carry):
                w0r = w0v[c * _CCHUNK + r, :]
                w1r = w1v[c * _CCHUNK + r, :]
                for j in range(D // 16):
                    sl = pl.ds(j * 16, 16)
                    buf_o[slot, r, sl] = (buf_a[slot, r, sl] * w0r
                                          + buf_b[slot, r, sl] * w1r)
                return carry

            lax.fori_loop(0, _CCHUNK, _row_add, 0)

            if wr is not None:
                wr.wait()
            wr = pltpu.async_copy(
                buf_o.at[slot], y_hbm.at[pl.ds(base + c * _CCHUNK, _CCHUNK)],
                sem_o)
            cps = nxt
        wr.wait()

    return sc_combine


def _expert_block(be_ref, x_ref, idx_ref, w1_ref, w2_ref,
                  out_ref, dest_out_ref, x16_ref, w1c_ref, w2c_ref, dst_s):
    b = pl.program_id(0)

    @pl.when(b == 0)
    def _():
        x16_ref[...] = x_ref[...].astype(jnp.bfloat16)
        idx = idx_ref[0]
        eids = lax.broadcasted_iota(jnp.int32, (E, 1), 0)
        oh0 = (idx[0:1, :] == eids).astype(jnp.int32)
        oh1 = (idx[1:2, :] == eids).astype(jnp.int32)
        g = oh0 + oh1
        s_inc = g
        sh = 1
        while sh < T:
            s_inc = s_inc + jnp.pad(s_inc, ((0, 0), (sh, 0)))[:, :T]
            sh *= 2
        s_exc = s_inc - g
        counts = s_inc[:, T - 1:T]
        padded = ((counts + B - 1) // B) * B
        pc = padded
        sh = 1
        while sh < E:
            pc = pc + jnp.pad(pc, ((sh, 0), (0, 0)))[:E, :]
            sh *= 2
        pstart = pc - padded
        d0 = (jnp.sum((s_exc + pstart) * oh0, axis=0, keepdims=True))
        d1 = (jnp.sum((s_exc + pstart) * oh1, axis=0, keepdims=True))
        dst = jnp.concatenate([d0, d1], axis=0)
        dst_s[...] = dst
        dest_out_ref[0] = dst

    @pl.when(b < be_ref[NB])
    def _():
        prev = be_ref[jnp.maximum(b - 1, 0)]
        changed = jnp.logical_or(b == 0, be_ref[b] != prev)

        @pl.when(changed)
        def _():
            w1c_ref[...] = w1_ref[0].astype(jnp.bfloat16)
            w2c_ref[...] = w2_ref[0].astype(jnp.bfloat16)

        rowid = B * b + lax.broadcasted_iota(jnp.int32, (B, 1), 0)
        d0 = dst_s[0:1, :]
        d1 = dst_s[1:2, :]
        p = ((rowid == d0) | (rowid == d1)).astype(jnp.bfloat16)
        xb = jnp.dot(p, x16_ref[...],
                     preferred_element_type=jnp.float32).astype(jnp.bfloat16)
        h = jnp.dot(xb, w1c_ref[...], preferred_element_type=jnp.float32)
        up = h[:, :H]
        gate = h[:, H:]
        act = up * (gate * jax.lax.logistic(gate))
        out_ref[...] = jnp.dot(act.astype(jnp.bfloat16), w2c_ref[...],
                               preferred_element_type=jnp.float32)


def _tc_experts(block_expert, x, idx3, W1, W2):
    grid_spec = pltpu.PrefetchScalarGridSpec(
        num_scalar_prefetch=1,
        grid=(NB,),
        in_specs=[
            pl.BlockSpec((T, D), lambda b, be: (0, 0)),
            pl.BlockSpec((1, K, T), lambda b, be: (0, 0, 0)),
            pl.BlockSpec((1, D, 2 * H), lambda b, be: (be[b], 0, 0)),
            pl.BlockSpec((1, H, D), lambda b, be: (be[b], 0, 0)),
        ],
        out_specs=[
            pl.BlockSpec((B, D), lambda b, be: (b, 0)),
            pl.BlockSpec((1, K, T), lambda b, be: (0, 0, 0)),
        ],
        scratch_shapes=[
            pltpu.VMEM((T, D), jnp.bfloat16),
            pltpu.VMEM((D, 2 * H), jnp.bfloat16),
            pltpu.VMEM((H, D), jnp.bfloat16),
            pltpu.VMEM((K, T), jnp.int32),
        ],
    )
    return pl.pallas_call(
        _expert_block,
        grid_spec=grid_spec,
        out_shape=[jax.ShapeDtypeStruct((NPAD, D), jnp.float32),
                   jax.ShapeDtypeStruct((1, K, T), jnp.int32)],
    )(block_expert, x, idx3, W1, W2)


def _block_experts(indices):
    e_flat = indices.reshape(TK).astype(jnp.int32)
    onehot = (e_flat[:, None] == jnp.arange(E, dtype=jnp.int32)[None, :])
    counts = jnp.sum(onehot.astype(jnp.int32), axis=0)
    padded_ends = jnp.cumsum(((counts + B - 1) // B) * B)
    block_expert = jnp.searchsorted(
        padded_ends, jnp.arange(NB, dtype=jnp.int32) * B, side="right")
    block_expert = jnp.minimum(block_expert, E - 1).astype(jnp.int32)
    nactive = padded_ends[E - 1] // B
    return jnp.concatenate([block_expert, nactive[None]])


def kernel(x, weights, indices, W1, W2):
    block_expert = _block_experts(indices)
    idx3 = indices.astype(jnp.int32).T.reshape(1, K, T)
    out_sorted, dest_out = _tc_experts(block_expert, x, idx3, W1, W2)
    wb = jnp.broadcast_to(weights.T.reshape(K, T, 1), (K, T, 16))
    return _sc_combine()(out_sorted, dest_out, wb)


# speedup vs baseline: 1.2151x; 1.0480x over previous
import functools

import jax
import jax.numpy as jnp
from jax import lax
from jax.experimental import pallas as pl
from jax.experimental.pallas import tpu as pltpu
from jax.experimental.pallas import tpu_sc as plsc

E = 8
D = 1024
H = 512
T = 2048
K = 2
TK = T * K

B = 256
NB = (TK + E * (B - 1) + B - 1) // B
NPAD = NB * B

NC = 2
NS = 16
NW = NC * NS

_CTOK = T // NW
_CCHUNK = 16


@functools.cache
def _sc_combine():
    mesh = plsc.VectorSubcoreMesh(core_axis_name="c", subcore_axis_name="s")

    @functools.partial(
        pl.kernel,
        out_type=jax.ShapeDtypeStruct((T, D), jnp.float32),
        mesh=mesh,
        scratch_types=[
            pltpu.VMEM((_CTOK,), jnp.int32),
            pltpu.VMEM((_CTOK,), jnp.int32),
            pltpu.VMEM((_CTOK, 16), jnp.float32),
            pltpu.VMEM((_CTOK, 16), jnp.float32),
            pltpu.VMEM((2, _CCHUNK, D), jnp.float32),
            pltpu.VMEM((2, _CCHUNK, D), jnp.float32),
            pltpu.VMEM((2, _CCHUNK, D), jnp.float32),
            pltpu.SemaphoreType.DMA((2,)),
            pltpu.SemaphoreType.DMA((2,)),
            pltpu.SemaphoreType.DMA,
        ],
    )
    def sc_combine(rows_hbm, pos_hbm, w_hbm, y_hbm,
                   idx_a, idx_b, w0v, w1v, buf_a, buf_b, buf_o,
                   sem_a, sem_b, sem_o):
        wid = lax.axis_index("s") * NC + lax.axis_index("c")
        base = wid * _CTOK
        nch = _CTOK // _CCHUNK
        pltpu.sync_copy(pos_hbm.at[0, 0, pl.ds(base, _CTOK)], idx_a)
        pltpu.sync_copy(pos_hbm.at[0, 1, pl.ds(base, _CTOK)], idx_b)
        pltpu.sync_copy(w_hbm.at[0, pl.ds(base, _CTOK)], w0v)
        pltpu.sync_copy(w_hbm.at[1, pl.ds(base, _CTOK)], w1v)

        def fire(c):
            slot = c % 2
            ia = idx_a.at[pl.ds(c * _CCHUNK, _CCHUNK)]
            ib = idx_b.at[pl.ds(c * _CCHUNK, _CCHUNK)]
            return (pltpu.async_copy(rows_hbm.at[ia], buf_a.at[slot],
                                     sem_a.at[slot]),
                    pltpu.async_copy(rows_hbm.at[ib], buf_b.at[slot],
                                     sem_b.at[slot]))

        cps = fire(0)
        wr = None
        for c in range(nch):
            slot = c % 2
            nxt = fire(c + 1) if c + 1 < nch else None
            cps[0].wait()
            cps[1].wait()

            def _row_add(r, carry):
                w0r = w0v[c * _CCHUNK + r, :]
                w1r = w1v[c * _CCHUNK + r, :]
                for j in range(D // 16):
                    sl = pl.ds(j * 16, 16)
                    buf_o[slot, r, sl] = (buf_a[slot, r, sl] * w0r
                                          + buf_b[slot, r, sl] * w1r)
                return carry

            lax.fori_loop(0, _CCHUNK, _row_add, 0)

            if wr is not None:
                wr.wait()
            wr = pltpu.async_copy(
                buf_o.at[slot], y_hbm.at[pl.ds(base + c * _CCHUNK, _CCHUNK)],
                sem_o)
            cps = nxt
        wr.wait()

    return sc_combine


def _expert_block(be_ref, x_ref, idx_ref, w1_ref, w2_ref,
                  out_ref, dest_out_ref, x16_ref, w1c_ref, w2c_ref, dst_s):
    b = pl.program_id(0)

    @pl.when(b == 0)
    def _():
        x16_ref[...] = x_ref[...].astype(jnp.bfloat16)
        idx = idx_ref[0]
        eids = lax.broadcasted_iota(jnp.int32, (E, 1), 0)
        oh0 = (idx[0:1, :] == eids).astype(jnp.int32)
        oh1 = (idx[1:2, :] == eids).astype(jnp.int32)
        g = oh0 + oh1
        s_inc = g
        sh = 1
        while sh < T:
            s_inc = s_inc + jnp.pad(s_inc, ((0, 0), (sh, 0)))[:, :T]
            sh *= 2
        s_exc = s_inc - g
        counts = s_inc[:, T - 1:T]
        padded = ((counts + B - 1) // B) * B
        pc = padded
        sh = 1
        while sh < E:
            pc = pc + jnp.pad(pc, ((sh, 0), (0, 0)))[:E, :]
            sh *= 2
        pstart = pc - padded
        d0 = (jnp.sum((s_exc + pstart) * oh0, axis=0, keepdims=True))
        d1 = (jnp.sum((s_exc + pstart) * oh1, axis=0, keepdims=True))
        dst = jnp.concatenate([d0, d1], axis=0)
        dst_s[...] = dst
        dest_out_ref[0] = dst

    @pl.when(b < be_ref[0, NB])
    def _():
        prev = be_ref[0, jnp.maximum(b - 1, 0)]
        changed = jnp.logical_or(b == 0, be_ref[0, b] != prev)

        @pl.when(changed)
        def _():
            w1c_ref[...] = w1_ref[0].astype(jnp.bfloat16)
            w2c_ref[...] = w2_ref[0].astype(jnp.bfloat16)

        rowid = B * b + lax.broadcasted_iota(jnp.int32, (B, 1), 0)
        d0 = dst_s[0:1, :]
        d1 = dst_s[1:2, :]
        p = ((rowid == d0) | (rowid == d1)).astype(jnp.bfloat16)
        xb = jnp.dot(p, x16_ref[...],
                     preferred_element_type=jnp.float32).astype(jnp.bfloat16)
        h = jnp.dot(xb, w1c_ref[...], preferred_element_type=jnp.float32)
        up = h[:, :H]
        gate = h[:, H:]
        act = up * (gate * jax.lax.logistic(gate))
        out_ref[...] = jnp.dot(act.astype(jnp.bfloat16), w2c_ref[...],
                               preferred_element_type=jnp.float32)


def _tc_experts(block_expert, x, idx3, W1, W2):
    grid_spec = pltpu.PrefetchScalarGridSpec(
        num_scalar_prefetch=1,
        grid=(NB,),
        in_specs=[
            pl.BlockSpec((T, D), lambda b, be: (0, 0)),
            pl.BlockSpec((1, K, T), lambda b, be: (0, 0, 0)),
            pl.BlockSpec((1, D, 2 * H), lambda b, be: (be[0, b], 0, 0)),
            pl.BlockSpec((1, H, D), lambda b, be: (be[0, b], 0, 0)),
        ],
        out_specs=[
            pl.BlockSpec((B, D), lambda b, be: (b, 0)),
            pl.BlockSpec((1, K, T), lambda b, be: (0, 0, 0)),
        ],
        scratch_shapes=[
            pltpu.VMEM((T, D), jnp.bfloat16),
            pltpu.VMEM((D, 2 * H), jnp.bfloat16),
            pltpu.VMEM((H, D), jnp.bfloat16),
            pltpu.VMEM((K, T), jnp.int32),
        ],
    )
    return pl.pallas_call(
        _expert_block,
        grid_spec=grid_spec,
        out_shape=[jax.ShapeDtypeStruct((NPAD, D), jnp.float32),
                   jax.ShapeDtypeStruct((1, K, T), jnp.int32)],
    )(block_expert, x, idx3, W1, W2)


def _be_body(idx_ref, be_ref):
    idx = idx_ref[0]
    eids = lax.broadcasted_iota(jnp.int32, (E, 1), 0)
    cnt = (jnp.sum((idx[0:1, :] == eids).astype(jnp.int32), axis=1,
                   keepdims=True)
           + jnp.sum((idx[1:2, :] == eids).astype(jnp.int32), axis=1,
                     keepdims=True))
    pe = ((cnt + B - 1) // B) * B
    sh = 1
    while sh < E:
        pe = pe + jnp.pad(pe, ((sh, 0), (0, 0)))[:E, :]
        sh *= 2
    bid = lax.broadcasted_iota(jnp.int32, (1, NB), 1) * B
    be = jnp.sum((pe <= bid).astype(jnp.int32), axis=0, keepdims=True)
    be = jnp.minimum(be, E - 1)
    nact = pe[E - 1:E, 0:1] // B
    be_ref[...] = jnp.concatenate([be, nact], axis=1)


def _block_experts(idx3):
    return pl.pallas_call(
        _be_body,
        out_shape=jax.ShapeDtypeStruct((1, NB + 1), jnp.int32),
    )(idx3)


def kernel(x, weights, indices, W1, W2):
    idx3 = indices.astype(jnp.int32).T.reshape(1, K, T)
    block_expert = _block_experts(idx3)
    out_sorted, dest_out = _tc_experts(block_expert, x, idx3, W1, W2)
    wb = jnp.broadcast_to(weights.T.reshape(K, T, 1), (K, T, 16))
    return _sc_combine()(out_sorted, dest_out, wb)


# speedup vs baseline: 1.2154x; 1.0002x over previous
import functools

import jax
import jax.numpy as jnp
from jax import lax
from jax.experimental import pallas as pl
from jax.experimental.pallas import tpu as pltpu
from jax.experimental.pallas import tpu_sc as plsc

E = 8
D = 1024
H = 512
T = 2048
K = 2
TK = T * K

B = 256
NB = (TK + E * (B - 1) + B - 1) // B
NPAD = NB * B

NC = 2
NS = 16
NW = NC * NS

_CTOK = T // NW
_CCHUNK = 16


@functools.cache
def _sc_combine():
    mesh = plsc.VectorSubcoreMesh(core_axis_name="c", subcore_axis_name="s")

    @functools.partial(
        pl.kernel,
        out_type=jax.ShapeDtypeStruct((T, D), jnp.float32),
        mesh=mesh,
        scratch_types=[
            pltpu.VMEM((_CTOK,), jnp.int32),
            pltpu.VMEM((_CTOK,), jnp.int32),
            pltpu.VMEM((_CTOK, 16), jnp.float32),
            pltpu.VMEM((_CTOK, 16), jnp.float32),
            pltpu.VMEM((2, _CCHUNK, D), jnp.float32),
            pltpu.VMEM((2, _CCHUNK, D), jnp.float32),
            pltpu.VMEM((2, _CCHUNK, D), jnp.float32),
            pltpu.SemaphoreType.DMA((2,)),
            pltpu.SemaphoreType.DMA((2,)),
            pltpu.SemaphoreType.DMA,
        ],
    )
    def sc_combine(rows_hbm, pos_hbm, w_hbm, y_hbm,
                   idx_a, idx_b, w0v, w1v, buf_a, buf_b, buf_o,
                   sem_a, sem_b, sem_o):
        wid = lax.axis_index("s") * NC + lax.axis_index("c")
        base = wid * _CTOK
        nch = _CTOK // _CCHUNK
        pltpu.sync_copy(pos_hbm.at[0, 0, pl.ds(base, _CTOK)], idx_a)
        pltpu.sync_copy(pos_hbm.at[0, 1, pl.ds(base, _CTOK)], idx_b)
        pltpu.sync_copy(w_hbm.at[0, pl.ds(base, _CTOK)], w0v)
        pltpu.sync_copy(w_hbm.at[1, pl.ds(base, _CTOK)], w1v)

        def fire(c):
            slot = c % 2
            ia = idx_a.at[pl.ds(c * _CCHUNK, _CCHUNK)]
            ib = idx_b.at[pl.ds(c * _CCHUNK, _CCHUNK)]
            return (pltpu.async_copy(rows_hbm.at[ia], buf_a.at[slot],
                                     sem_a.at[slot]),
                    pltpu.async_copy(rows_hbm.at[ib], buf_b.at[slot],
                                     sem_b.at[slot]))

        cps = fire(0)
        wr = None
        for c in range(nch):
            slot = c % 2
            nxt = fire(c + 1) if c + 1 < nch else None
            cps[0].wait()
            cps[1].wait()

            def _row_add(r, carry):
                w0r = w0v[c * _CCHUNK + r, :]
                w1r = w1v[c * _CCHUNK + r, :]
                for j in range(D // 16):
                    sl = pl.ds(j * 16, 16)
                    buf_o[slot, r, sl] = (buf_a[slot, r, sl] * w0r
                                          + buf_b[slot, r, sl] * w1r)
                return carry

            lax.fori_loop(0, _CCHUNK, _row_add, 0)

            if wr is not None:
                wr.wait()
            wr = pltpu.async_copy(
                buf_o.at[slot], y_hbm.at[pl.ds(base + c * _CCHUNK, _CCHUNK)],
                sem_o)
            cps = nxt
        wr.wait()

    return sc_combine


def _expert_block(be_ref, x_ref, idx_ref, w1_ref, w2_ref,
                  out_ref, dest_out_ref, x16_ref, w1c_ref, w2c_ref, dst_s):
    b = pl.program_id(0)

    @pl.when(b == 0)
    def _():
        x16_ref[...] = x_ref[...].astype(jnp.bfloat16)
        idx = idx_ref[0]
        eids = lax.broadcasted_iota(jnp.int32, (E, 1), 0)
        oh0 = (idx[0:1, :] == eids).astype(jnp.int32)
        oh1 = (idx[1:2, :] == eids).astype(jnp.int32)
        g = oh0 + oh1
        s_inc = g
        sh = 1
        while sh < T:
            s_inc = s_inc + jnp.pad(s_inc, ((0, 0), (sh, 0)))[:, :T]
            sh *= 2
        s_exc = s_inc - g
        counts = s_inc[:, T - 1:T]
        padded = ((counts + B - 1) // B) * B
        pc = padded
        sh = 1
        while sh < E:
            pc = pc + jnp.pad(pc, ((sh, 0), (0, 0)))[:E, :]
            sh *= 2
        pstart = pc - padded
        d0 = (jnp.sum((s_exc + pstart) * oh0, axis=0, keepdims=True))
        d1 = (jnp.sum((s_exc + pstart) * oh1, axis=0, keepdims=True))
        dst = jnp.concatenate([d0, d1], axis=0)
        dst_s[...] = dst.astype(jnp.int16)
        dest_out_ref[0] = dst

    @pl.when(b < be_ref[0, NB])
    def _():
        prev = be_ref[0, jnp.maximum(b - 1, 0)]
        changed = jnp.logical_or(b == 0, be_ref[0, b] != prev)

        @pl.when(changed)
        def _():
            w1c_ref[...] = w1_ref[0].astype(jnp.bfloat16)
            w2c_ref[...] = w2_ref[0].astype(jnp.bfloat16)

        rowid = (B * b
                 + lax.broadcasted_iota(jnp.int32, (B, 1), 0)).astype(jnp.int16)
        d0 = dst_s[0:1, :]
        d1 = dst_s[1:2, :]
        p = ((rowid == d0) | (rowid == d1)).astype(jnp.bfloat16)
        xb = jnp.dot(p, x16_ref[...],
                     preferred_element_type=jnp.float32).astype(jnp.bfloat16)
        h = jnp.dot(xb, w1c_ref[...], preferred_element_type=jnp.float32)
        up = h[:, :H]
        gate = h[:, H:]
        act = up * (gate * jax.lax.logistic(gate))
        out_ref[...] = jnp.dot(act.astype(jnp.bfloat16), w2c_ref[...],
                               preferred_element_type=jnp.float32)


def _tc_experts(block_expert, x, idx3, W1, W2):
    grid_spec = pltpu.PrefetchScalarGridSpec(
        num_scalar_prefetch=1,
        grid=(NB,),
        in_specs=[
            pl.BlockSpec((T, D), lambda b, be: (0, 0)),
            pl.BlockSpec((1, K, T), lambda b, be: (0, 0, 0)),
            pl.BlockSpec((1, D, 2 * H), lambda b, be: (be[0, b], 0, 0)),
            pl.BlockSpec((1, H, D), lambda b, be: (be[0, b], 0, 0)),
        ],
        out_specs=[
            pl.BlockSpec((B, D), lambda b, be: (b, 0)),
            pl.BlockSpec((1, K, T), lambda b, be: (0, 0, 0)),
        ],
        scratch_shapes=[
            pltpu.VMEM((T, D), jnp.bfloat16),
            pltpu.VMEM((D, 2 * H), jnp.bfloat16),
            pltpu.VMEM((H, D), jnp.bfloat16),
            pltpu.VMEM((K, T), jnp.int16),
        ],
    )
    return pl.pallas_call(
        _expert_block,
        grid_spec=grid_spec,
        out_shape=[jax.ShapeDtypeStruct((NPAD, D), jnp.float32),
                   jax.ShapeDtypeStruct((1, K, T), jnp.int32)],
    )(block_expert, x, idx3, W1, W2)


def _be_body(idx_ref, be_ref):
    idx = idx_ref[0]
    eids = lax.broadcasted_iota(jnp.int32, (E, 1), 0)
    cnt = (jnp.sum((idx[0:1, :] == eids).astype(jnp.int32), axis=1,
                   keepdims=True)
           + jnp.sum((idx[1:2, :] == eids).astype(jnp.int32), axis=1,
                     keepdims=True))
    pe = ((cnt + B - 1) // B) * B
    sh = 1
    while sh < E:
        pe = pe + jnp.pad(pe, ((sh, 0), (0, 0)))[:E, :]
        sh *= 2
    bid = lax.broadcasted_iota(jnp.int32, (1, NB), 1) * B
    be = jnp.sum((pe <= bid).astype(jnp.int32), axis=0, keepdims=True)
    be = jnp.minimum(be, E - 1)
    nact = pe[E - 1:E, 0:1] // B
    be_ref[...] = jnp.concatenate([be, nact], axis=1)


def _block_experts(idx3):
    return pl.pallas_call(
        _be_body,
        out_shape=jax.ShapeDtypeStruct((1, NB + 1), jnp.int32),
    )(idx3)


def kernel(x, weights, indices, W1, W2):
    idx3 = indices.astype(jnp.int32).T.reshape(1, K, T)
    block_expert = _block_experts(idx3)
    out_sorted, dest_out = _tc_experts(block_expert, x, idx3, W1, W2)
    wb = jnp.broadcast_to(weights.T.reshape(K, T, 1), (K, T, 16))
    return _sc_combine()(out_sorted, dest_out, wb)
